# baseline (speedup 1.0000x reference)
import jax
import jax.numpy as jnp
from jax.experimental import pallas as pl
from jax.experimental.pallas import tpu as pltpu
from jax.experimental.pallas import tpu_sc as plsc

N = 32768
IN_DIM = 1024
HD = 256
K1 = 8
BLK = 1024
NBLK = N // BLK
LANES = 16
NCH = BLK // LANES
NWORK = 32


def _dot(a, b):
    return jax.lax.dot_general(a.astype(jnp.bfloat16), b.astype(jnp.bfloat16),
                               (((1,), (0,)), ((), ())),
                               preferred_element_type=jnp.float32)


def _fwd_body(h_ref, wfc_ref, bfc_ref, wab_ref, bab_ref, wc_ref, bc_ref,
              x_ref, a_ref):
    x = jnp.maximum(_dot(h_ref[:], wfc_ref[:]) + bfc_ref[:], 0.0)
    xb = x.astype(jnp.bfloat16)
    x_ref[:] = xb
    t = jnp.tanh(_dot(xb, wab_ref[:]) + bab_ref[:])
    g1 = t[:, 0 * HD:1 * HD] * (0.5 * t[:, 1 * HD:2 * HD] + 0.5)
    g2 = t[:, 2 * HD:3 * HD] * (0.5 * t[:, 3 * HD:4 * HD] + 0.5)
    a12 = _dot(jnp.concatenate([g1, g2], axis=1), wc_ref[:]) + bc_ref[:]
    a_ref[:] = jnp.transpose(a12, (1, 0))


def _sc_topk(a12, lane_iota):
    mesh = plsc.VectorSubcoreMesh(core_axis_name="c", subcore_axis_name="s")

    @pl.kernel(
        out_type=(jax.ShapeDtypeStruct((NWORK, K1, LANES), jnp.float32),
                  jax.ShapeDtypeStruct((NWORK, K1, LANES), jnp.int32)),
        mesh=mesh,
        scratch_types=[pltpu.VMEM((BLK,), jnp.float32),
                       pltpu.VMEM((LANES,), jnp.int32),
                       pltpu.VMEM((K1, LANES), jnp.float32),
                       pltpu.VMEM((K1, LANES), jnp.int32),
                       pltpu.SemaphoreType.DMA],
    )
    def sel(a_hbm, lane_hbm, vals_hbm, idx_hbm, chunk, lane, vtop, itop, sem):
        ci = jax.lax.axis_index("c")
        si = jax.lax.axis_index("s")
        wkr = ci * 16 + si
        pltpu.async_copy(a_hbm.at[wkr, 0], chunk, sem).wait()
        pltpu.async_copy(lane_hbm, lane, sem).wait()

        neg = jnp.float32(-3e38)
        for r in range(K1):
            vtop[r, :] = jnp.full((LANES,), neg, jnp.float32)
            itop[r, :] = jnp.zeros((LANES,), jnp.int32)

        base = wkr * BLK

        @pl.loop(0, NCH)
        def _(j):
            x = chunk[pl.ds(j * LANES, LANES)]
            fl = base + j * LANES + lane[:]
            for r in range(K1):
                cv = vtop[r, :]
                civ = itop[r, :]
                gt = x > cv
                vtop[r, :] = jnp.where(gt, x, cv)
                itop[r, :] = jnp.where(gt, fl, civ)
                x = jnp.where(gt, cv, x)
                fl = jnp.where(gt, civ, fl)

        pltpu.async_copy(vtop, vals_hbm.at[wkr], sem).wait()
        pltpu.async_copy(itop, idx_hbm.at[wkr], sem).wait()

    return sel(a12, lane_iota)


def _pool_body(cv_ref, ci_ref, cx_ref, cy_ref, a_ref, x_ref,
               wcls_ref, bcls_ref, out_ref, w_scr, acc_ref):
    i = pl.program_id(0)

    @pl.when(i == 0)
    def _():
        acc_ref[:] = jnp.zeros_like(acc_ref)
        rows = jax.lax.broadcasted_iota(jnp.int32, (NBLK, BLK), 0)
        cols = jax.lax.broadcasted_iota(jnp.int32, (NBLK, BLK), 1)
        flat = rows * BLK + cols
        cx = cx_ref[:]
        cy = cy_ref[:]
        cv = cv_ref[:]
        civ = ci_ref[:]
        big = jnp.float32(1e30)
        neg = jnp.float32(-3e38)
        xmin = big
        xmax = -big
        ymin = big
        ymax = -big
        for _ in range(K1):
            m = jnp.max(cv)
            sidx = jnp.min(jnp.where(cv >= m, civ, jnp.int32(2**30)))
            sel = flat == sidx
            xmin = jnp.minimum(xmin, jnp.min(jnp.where(sel, cx, big)))
            xmax = jnp.maximum(xmax, jnp.max(jnp.where(sel, cx, -big)))
            ymin = jnp.minimum(ymin, jnp.min(jnp.where(sel, cy, big)))
            ymax = jnp.maximum(ymax, jnp.max(jnp.where(sel, cy, -big)))
            cv = jnp.where(civ == sidx, neg, cv)

        inb = ((cx >= xmin) & (cx <= xmax) & (cy >= ymin) & (cy <= ymax))
        a2v = a_ref[:, 1, :]
        mx = jnp.max(jnp.where(inb, a2v, -big))
        e = jnp.where(inb, jnp.exp(a2v - mx), 0.0)
        z = jnp.sum(e)
        w_scr[:] = e / z

    acc_ref[:] += jax.lax.dot_general(
        w_scr[pl.ds(i, 1), :].astype(jnp.bfloat16), x_ref[:],
        (((1,), (0,)), ((), ())), preferred_element_type=jnp.float32)

    @pl.when(i == NBLK - 1)
    def _():
        out_ref[:] = _dot(acc_ref[:], wcls_ref[:]) + bcls_ref[:]


@jax.jit
def kernel(h, coords, W_fc, b_fc, Wa1, ba1, Wb1, bb1, Wc1, bc1,
           Wa2, ba2, Wb2, bb2, Wc2, bc2, W_cls, b_cls):
    f32 = jnp.float32
    full = lambda *s: pl.BlockSpec(s, lambda i: tuple(0 for _ in s))

    Wab = jnp.concatenate([Wa1, 0.5 * Wb1, Wa2, 0.5 * Wb2], axis=1)
    bab = jnp.concatenate([ba1, 0.5 * bb1, ba2, 0.5 * bb2]).reshape(1, 4 * HD)
    z = jnp.zeros((HD, 1), f32)
    Wc = jnp.concatenate(
        [jnp.concatenate([Wc1, z], axis=1),
         jnp.concatenate([z, Wc2], axis=1)], axis=0)
    bc = jnp.stack([bc1[0], bc2[0]]).reshape(1, 2)

    x, a12 = pl.pallas_call(
        _fwd_body,
        grid=(NBLK,),
        in_specs=[
            pl.BlockSpec((BLK, IN_DIM), lambda i: (i, 0)),
            full(IN_DIM, HD), full(1, HD),
            full(HD, 4 * HD), full(1, 4 * HD),
            full(2 * HD, 2), full(1, 2),
        ],
        out_specs=(
            pl.BlockSpec((BLK, HD), lambda i: (i, 0)),
            pl.BlockSpec((None, 2, BLK), lambda i: (i, 0, 0)),
        ),
        out_shape=(
            jax.ShapeDtypeStruct((N, HD), jnp.bfloat16),
            jax.ShapeDtypeStruct((NBLK, 2, BLK), f32),
        ),
        compiler_params=pltpu.CompilerParams(
            dimension_semantics=("arbitrary",)),
    )(h, W_fc, b_fc.reshape(1, HD), Wab, bab, Wc, bc)

    lane_iota = jnp.arange(LANES, dtype=jnp.int32)
    cand_v, cand_i = _sc_topk(a12, lane_iota)

    logits = pl.pallas_call(
        _pool_body,
        grid=(NBLK,),
        in_specs=[
            full(NWORK * K1, LANES), full(NWORK * K1, LANES),
            full(NBLK, BLK), full(NBLK, BLK),
            full(NBLK, 2, BLK),
            pl.BlockSpec((BLK, HD), lambda i: (i, 0)),
            full(HD, 2), full(1, 2),
        ],
        out_specs=pl.BlockSpec((1, 2), lambda i: (0, 0)),
        out_shape=jax.ShapeDtypeStruct((1, 2), f32),
        scratch_shapes=[
            pltpu.VMEM((NBLK, BLK), f32),
            pltpu.VMEM((1, HD), f32),
        ],
        compiler_params=pltpu.CompilerParams(
            dimension_semantics=("arbitrary",)),
    )(cand_v.reshape(NWORK * K1, LANES), cand_i.reshape(NWORK * K1, LANES),
      coords[:, 0].reshape(NBLK, BLK), coords[:, 1].reshape(NBLK, BLK),
      a12, x, W_cls, b_cls.reshape(1, 2))

    return logits


# speedup vs baseline: 1.1329x; 1.1329x over previous
import jax
import jax.numpy as jnp
from jax.experimental import pallas as pl
from jax.experimental.pallas import tpu as pltpu
from jax.experimental.pallas import tpu_sc as plsc

N = 32768
IN_DIM = 1024
HD = 256
K1 = 8
BLK = 2048
NBLK = N // BLK
LANES = 16
NWORK = 32
WCHUNK = N // NWORK
NCH = WCHUNK // LANES
WPR = BLK // WCHUNK


def _dot(a, b):
    return jax.lax.dot_general(a.astype(jnp.bfloat16), b.astype(jnp.bfloat16),
                               (((1,), (0,)), ((), ())),
                               preferred_element_type=jnp.float32)


def _fwd_body(h_ref, wfc_ref, bfc_ref, wab_ref, bab_ref, wc_ref, bc_ref,
              x_ref, a_ref):
    x = jnp.maximum(_dot(h_ref[:], wfc_ref[:]) + bfc_ref[:], 0.0)
    xb = x.astype(jnp.bfloat16)
    x_ref[:] = xb
    t = jnp.tanh(_dot(xb, wab_ref[:]) + bab_ref[:])
    g1 = t[:, 0 * HD:1 * HD] * (0.5 * t[:, 1 * HD:2 * HD] + 0.5)
    g2 = t[:, 2 * HD:3 * HD] * (0.5 * t[:, 3 * HD:4 * HD] + 0.5)
    a12 = _dot(jnp.concatenate([g1, g2], axis=1), wc_ref[:]) + bc_ref[:]
    a_ref[:] = jnp.transpose(a12, (1, 0))


def _sc_topk(a12, lane_iota):
    mesh = plsc.VectorSubcoreMesh(core_axis_name="c", subcore_axis_name="s")

    @pl.kernel(
        out_type=(jax.ShapeDtypeStruct((NWORK, K1, LANES), jnp.float32),
                  jax.ShapeDtypeStruct((NWORK, K1, LANES), jnp.int32)),
        mesh=mesh,
        scratch_types=[pltpu.VMEM((WCHUNK,), jnp.float32),
                       pltpu.VMEM((LANES,), jnp.int32),
                       pltpu.VMEM((K1, LANES), jnp.float32),
                       pltpu.VMEM((K1, LANES), jnp.int32),
                       pltpu.SemaphoreType.DMA],
    )
    def sel(a_hbm, lane_hbm, vals_hbm, idx_hbm, chunk, lane, vtop, itop, sem):
        ci = jax.lax.axis_index("c")
        si = jax.lax.axis_index("s")
        wkr = ci * 16 + si
        pltpu.async_copy(
            a_hbm.at[wkr // WPR, 0, pl.ds((wkr % WPR) * WCHUNK, WCHUNK)],
            chunk, sem).wait()
        pltpu.async_copy(lane_hbm, lane, sem).wait()

        neg = jnp.float32(-3e38)
        for r in range(K1):
            vtop[r, :] = jnp.full((LANES,), neg, jnp.float32)
            itop[r, :] = jnp.zeros((LANES,), jnp.int32)

        base = wkr * WCHUNK

        @pl.loop(0, NCH)
        def _(j):
            x = chunk[pl.ds(j * LANES, LANES)]
            fl = base + j * LANES + lane[:]
            for r in range(K1):
                cv = vtop[r, :]
                civ = itop[r, :]
                gt = x > cv
                vtop[r, :] = jnp.where(gt, x, cv)
                itop[r, :] = jnp.where(gt, fl, civ)
                x = jnp.where(gt, cv, x)
                fl = jnp.where(gt, civ, fl)

        pltpu.async_copy(vtop, vals_hbm.at[wkr], sem).wait()
        pltpu.async_copy(itop, idx_hbm.at[wkr], sem).wait()

    return sel(a12, lane_iota)


def _pool_body(cv_ref, ci_ref, cx_ref, cy_ref, a_ref, x_ref,
               wcls_ref, bcls_ref, out_ref, w_scr, acc_ref):
    i = pl.program_id(0)

    @pl.when(i == 0)
    def _():
        acc_ref[:] = jnp.zeros_like(acc_ref)
        rows = jax.lax.broadcasted_iota(jnp.int32, (NBLK, BLK), 0)
        cols = jax.lax.broadcasted_iota(jnp.int32, (NBLK, BLK), 1)
        flat = rows * BLK + cols
        cx = cx_ref[:]
        cy = cy_ref[:]
        cv = cv_ref[:]
        civ = ci_ref[:]
        big = jnp.float32(1e30)
        neg = jnp.float32(-3e38)
        xmin = big
        xmax = -big
        ymin = big
        ymax = -big
        for _ in range(K1):
            m = jnp.max(cv)
            sidx = jnp.min(jnp.where(cv >= m, civ, jnp.int32(2**30)))
            sel = flat == sidx
            xmin = jnp.minimum(xmin, jnp.min(jnp.where(sel, cx, big)))
            xmax = jnp.maximum(xmax, jnp.max(jnp.where(sel, cx, -big)))
            ymin = jnp.minimum(ymin, jnp.min(jnp.where(sel, cy, big)))
            ymax = jnp.maximum(ymax, jnp.max(jnp.where(sel, cy, -big)))
            cv = jnp.where(civ == sidx, neg, cv)

        inb = ((cx >= xmin) & (cx <= xmax) & (cy >= ymin) & (cy <= ymax))
        a2v = a_ref[:, 1, :]
        mx = jnp.max(jnp.where(inb, a2v, -big))
        e = jnp.where(inb, jnp.exp(a2v - mx), 0.0)
        z = jnp.sum(e)
        w_scr[:] = e / z

    acc_ref[:] += jax.lax.dot_general(
        w_scr[pl.ds(i, 1), :].astype(jnp.bfloat16), x_ref[:],
        (((1,), (0,)), ((), ())), preferred_element_type=jnp.float32)

    @pl.when(i == NBLK - 1)
    def _():
        out_ref[:] = _dot(acc_ref[:], wcls_ref[:]) + bcls_ref[:]


@jax.jit
def kernel(h, coords, W_fc, b_fc, Wa1, ba1, Wb1, bb1, Wc1, bc1,
           Wa2, ba2, Wb2, bb2, Wc2, bc2, W_cls, b_cls):
    f32 = jnp.float32
    full = lambda *s: pl.BlockSpec(s, lambda i: tuple(0 for _ in s))

    Wab = jnp.concatenate([Wa1, 0.5 * Wb1, Wa2, 0.5 * Wb2], axis=1)
    bab = jnp.concatenate([ba1, 0.5 * bb1, ba2, 0.5 * bb2]).reshape(1, 4 * HD)
    z = jnp.zeros((HD, 1), f32)
    Wc = jnp.concatenate(
        [jnp.concatenate([Wc1, z], axis=1),
         jnp.concatenate([z, Wc2], axis=1)], axis=0)
    bc = jnp.stack([bc1[0], bc2[0]]).reshape(1, 2)

    x, a12 = pl.pallas_call(
        _fwd_body,
        grid=(NBLK,),
        in_specs=[
            pl.BlockSpec((BLK, IN_DIM), lambda i: (i, 0)),
            full(IN_DIM, HD), full(1, HD),
            full(HD, 4 * HD), full(1, 4 * HD),
            full(2 * HD, 2), full(1, 2),
        ],
        out_specs=(
            pl.BlockSpec((BLK, HD), lambda i: (i, 0)),
            pl.BlockSpec((None, 2, BLK), lambda i: (i, 0, 0)),
        ),
        out_shape=(
            jax.ShapeDtypeStruct((N, HD), jnp.bfloat16),
            jax.ShapeDtypeStruct((NBLK, 2, BLK), f32),
        ),
        compiler_params=pltpu.CompilerParams(
            dimension_semantics=("arbitrary",)),
    )(h, W_fc, b_fc.reshape(1, HD), Wab, bab, Wc, bc)

    lane_iota = jnp.arange(LANES, dtype=jnp.int32)
    cand_v, cand_i = _sc_topk(a12, lane_iota)

    logits = pl.pallas_call(
        _pool_body,
        grid=(NBLK,),
        in_specs=[
            full(NWORK, K1, LANES), full(NWORK, K1, LANES),
            full(NBLK, BLK), full(NBLK, BLK),
            full(NBLK, 2, BLK),
            pl.BlockSpec((BLK, HD), lambda i: (i, 0)),
            full(HD, 2), full(1, 2),
        ],
        out_specs=pl.BlockSpec((1, 2), lambda i: (0, 0)),
        out_shape=jax.ShapeDtypeStruct((1, 2), f32),
        scratch_shapes=[
            pltpu.VMEM((NBLK, BLK), f32),
            pltpu.VMEM((1, HD), f32),
        ],
        compiler_params=pltpu.CompilerParams(
            dimension_semantics=("arbitrary",)),
    )(cand_v, cand_i,
      coords[:, 0].reshape(NBLK, BLK), coords[:, 1].reshape(NBLK, BLK),
      a12, x, W_cls, b_cls.reshape(1, 2))

    return logits


# speedup vs baseline: 1.2013x; 1.0604x over previous
import jax
import jax.numpy as jnp
from jax.experimental import pallas as pl
from jax.experimental.pallas import tpu as pltpu
from jax.experimental.pallas import tpu_sc as plsc

N = 32768
IN_DIM = 1024
HD = 256
K1 = 8
BLK = 4096
NBLK = N // BLK
LANES = 16
NWORK = 32
WCHUNK = N // NWORK
NCH = WCHUNK // LANES
WPR = BLK // WCHUNK


def _dot(a, b):
    return jax.lax.dot_general(a.astype(jnp.bfloat16), b.astype(jnp.bfloat16),
                               (((1,), (0,)), ((), ())),
                               preferred_element_type=jnp.float32)


def _fwd_body(h_ref, wfc_ref, bfc_ref, wab_ref, bab_ref, wc_ref, bc_ref,
              x_ref, a_ref):
    x = jnp.maximum(_dot(h_ref[:], wfc_ref[:]) + bfc_ref[:], 0.0)
    xb = x.astype(jnp.bfloat16)
    x_ref[:] = xb
    t = jnp.tanh(_dot(xb, wab_ref[:]) + bab_ref[:])
    g1 = t[:, 0 * HD:1 * HD] * (0.5 * t[:, 1 * HD:2 * HD] + 0.5)
    g2 = t[:, 2 * HD:3 * HD] * (0.5 * t[:, 3 * HD:4 * HD] + 0.5)
    a12 = _dot(jnp.concatenate([g1, g2], axis=1), wc_ref[:]) + bc_ref[:]
    a_ref[:] = jnp.transpose(a12, (1, 0))


def _sc_topk(a12, lane_iota):
    mesh = plsc.VectorSubcoreMesh(core_axis_name="c", subcore_axis_name="s")

    @pl.kernel(
        out_type=(jax.ShapeDtypeStruct((NWORK, K1, LANES), jnp.float32),
                  jax.ShapeDtypeStruct((NWORK, K1, LANES), jnp.int32)),
        mesh=mesh,
        scratch_types=[pltpu.VMEM((WCHUNK,), jnp.float32),
                       pltpu.VMEM((LANES,), jnp.int32),
                       pltpu.VMEM((K1, LANES), jnp.float32),
                       pltpu.VMEM((K1, LANES), jnp.int32),
                       pltpu.SemaphoreType.DMA],
    )
    def sel(a_hbm, lane_hbm, vals_hbm, idx_hbm, chunk, lane, vtop, itop, sem):
        ci = jax.lax.axis_index("c")
        si = jax.lax.axis_index("s")
        wkr = ci * 16 + si
        pltpu.async_copy(
            a_hbm.at[wkr // WPR, 0, pl.ds((wkr % WPR) * WCHUNK, WCHUNK)],
            chunk, sem).wait()
        pltpu.async_copy(lane_hbm, lane, sem).wait()

        neg = jnp.float32(-3e38)
        for r in range(K1):
            vtop[r, :] = jnp.full((LANES,), neg, jnp.float32)
            itop[r, :] = jnp.zeros((LANES,), jnp.int32)

        base = wkr * WCHUNK

        @pl.loop(0, NCH)
        def _(j):
            x = chunk[pl.ds(j * LANES, LANES)]
            fl = base + j * LANES + lane[:]
            for r in range(K1):
                cv = vtop[r, :]
                civ = itop[r, :]
                gt = x > cv
                vtop[r, :] = jnp.where(gt, x, cv)
                itop[r, :] = jnp.where(gt, fl, civ)
                x = jnp.where(gt, cv, x)
                fl = jnp.where(gt, civ, fl)

        pltpu.async_copy(vtop, vals_hbm.at[wkr], sem).wait()
        pltpu.async_copy(itop, idx_hbm.at[wkr], sem).wait()

    return sel(a12, lane_iota)


def _pool_body(cv_ref, ci_ref, cx_ref, cy_ref, a_ref, x_ref,
               wcls_ref, bcls_ref, out_ref, w_scr, acc_ref):
    i = pl.program_id(0)

    @pl.when(i == 0)
    def _():
        acc_ref[:] = jnp.zeros_like(acc_ref)
        rows = jax.lax.broadcasted_iota(jnp.int32, (NBLK, BLK), 0)
        cols = jax.lax.broadcasted_iota(jnp.int32, (NBLK, BLK), 1)
        flat = rows * BLK + cols
        cx = cx_ref[:]
        cy = cy_ref[:]
        cv = cv_ref[:]
        civ = ci_ref[:]
        big = jnp.float32(1e30)
        neg = jnp.float32(-3e38)
        xmin = big
        xmax = -big
        ymin = big
        ymax = -big
        for _ in range(K1):
            m = jnp.max(cv)
            sidx = jnp.min(jnp.where(cv >= m, civ, jnp.int32(2**30)))
            sel = flat == sidx
            xmin = jnp.minimum(xmin, jnp.min(jnp.where(sel, cx, big)))
            xmax = jnp.maximum(xmax, jnp.max(jnp.where(sel, cx, -big)))
            ymin = jnp.minimum(ymin, jnp.min(jnp.where(sel, cy, big)))
            ymax = jnp.maximum(ymax, jnp.max(jnp.where(sel, cy, -big)))
            cv = jnp.where(civ == sidx, neg, cv)

        inb = ((cx >= xmin) & (cx <= xmax) & (cy >= ymin) & (cy <= ymax))
        a2v = a_ref[:, 1, :]
        mx = jnp.max(jnp.where(inb, a2v, -big))
        e = jnp.where(inb, jnp.exp(a2v - mx), 0.0)
        z = jnp.sum(e)
        w_scr[:] = e / z

    acc_ref[:] += jax.lax.dot_general(
        w_scr[pl.ds(i, 1), :].astype(jnp.bfloat16), x_ref[:],
        (((1,), (0,)), ((), ())), preferred_element_type=jnp.float32)

    @pl.when(i == NBLK - 1)
    def _():
        out_ref[:] = _dot(acc_ref[:], wcls_ref[:]) + bcls_ref[:]


@jax.jit
def kernel(h, coords, W_fc, b_fc, Wa1, ba1, Wb1, bb1, Wc1, bc1,
           Wa2, ba2, Wb2, bb2, Wc2, bc2, W_cls, b_cls):
    f32 = jnp.float32
    full = lambda *s: pl.BlockSpec(s, lambda i: tuple(0 for _ in s))

    Wab = jnp.concatenate([Wa1, 0.5 * Wb1, Wa2, 0.5 * Wb2], axis=1)
    bab = jnp.concatenate([ba1, 0.5 * bb1, ba2, 0.5 * bb2]).reshape(1, 4 * HD)
    z = jnp.zeros((HD, 1), f32)
    Wc = jnp.concatenate(
        [jnp.concatenate([Wc1, z], axis=1),
         jnp.concatenate([z, Wc2], axis=1)], axis=0)
    bc = jnp.stack([bc1[0], bc2[0]]).reshape(1, 2)

    x, a12 = pl.pallas_call(
        _fwd_body,
        grid=(NBLK,),
        in_specs=[
            pl.BlockSpec((BLK, IN_DIM), lambda i: (i, 0)),
            full(IN_DIM, HD), full(1, HD),
            full(HD, 4 * HD), full(1, 4 * HD),
            full(2 * HD, 2), full(1, 2),
        ],
        out_specs=(
            pl.BlockSpec((BLK, HD), lambda i: (i, 0)),
            pl.BlockSpec((None, 2, BLK), lambda i: (i, 0, 0)),
        ),
        out_shape=(
            jax.ShapeDtypeStruct((N, HD), jnp.bfloat16),
            jax.ShapeDtypeStruct((NBLK, 2, BLK), f32),
        ),
        compiler_params=pltpu.CompilerParams(
            dimension_semantics=("arbitrary",)),
    )(h, W_fc, b_fc.reshape(1, HD), Wab, bab, Wc, bc)

    lane_iota = jnp.arange(LANES, dtype=jnp.int32)
    cand_v, cand_i = _sc_topk(a12, lane_iota)

    logits = pl.pallas_call(
        _pool_body,
        grid=(NBLK,),
        in_specs=[
            full(NWORK, K1, LANES), full(NWORK, K1, LANES),
            full(NBLK, BLK), full(NBLK, BLK),
            full(NBLK, 2, BLK),
            pl.BlockSpec((BLK, HD), lambda i: (i, 0)),
            full(HD, 2), full(1, 2),
        ],
        out_specs=pl.BlockSpec((1, 2), lambda i: (0, 0)),
        out_shape=jax.ShapeDtypeStruct((1, 2), f32),
        scratch_shapes=[
            pltpu.VMEM((NBLK, BLK), f32),
            pltpu.VMEM((1, HD), f32),
        ],
        compiler_params=pltpu.CompilerParams(
            dimension_semantics=("arbitrary",)),
    )(cand_v, cand_i,
      coords[:, 0].reshape(NBLK, BLK), coords[:, 1].reshape(NBLK, BLK),
      a12, x, W_cls, b_cls.reshape(1, 2))

    return logits
